# baseline (speedup 1.0000x reference)
import jax
import jax.numpy as jnp
from jax import lax
from jax.experimental import pallas as pl
from jax.experimental.pallas import tpu as pltpu
from jax.experimental.pallas import tpu_sc as plsc

BATCH = 16384
EMB = 32
ROWS_PER_SUPER = 128 // EMB
NUM_CORES = 2
NUM_SUBCORES = 16
NUM_WORKERS = NUM_CORES * NUM_SUBCORES
BPW = BATCH // NUM_WORKERS
CHUNK = 128
NCHUNKS = BPW // CHUNK
LANES = 16


def _mf_body(ubig_hbm, ibig_hbm, uoff_hbm, ioff_hbm, utab_hbm, itab_hbm,
             out_hbm, ubig, ibig, uoff, ioff, ubuf, ibuf, outv, usem, isem):
    wid = lax.axis_index("s") * NUM_CORES + lax.axis_index("c")

    pltpu.sync_copy(ubig_hbm.at[wid], ubig)
    pltpu.sync_copy(ibig_hbm.at[wid], ibig)
    pltpu.sync_copy(uoff_hbm.at[wid], uoff)
    pltpu.sync_copy(ioff_hbm.at[wid], ioff)

    for c in range(NCHUNKS):
        ucp = pltpu.async_copy(utab_hbm.at[ubig.at[c]], ubuf, usem)
        icp = pltpu.async_copy(itab_hbm.at[ibig.at[c]], ibuf, isem)
        ucp.wait()
        icp.wait()

        @pl.loop(0, CHUNK, step=LANES)
        def _(i0):
            rows = lax.iota(jnp.int32, LANES) + i0
            ubase = uoff[pl.ds(c * CHUNK + i0, LANES)]
            ibase = ioff[pl.ds(c * CHUNK + i0, LANES)]
            acc = (plsc.load_gather(ubuf, [rows, ubase]) *
                   plsc.load_gather(ibuf, [rows, ibase]))
            for j in range(1, EMB):
                acc = acc + (plsc.load_gather(ubuf, [rows, ubase + j]) *
                             plsc.load_gather(ibuf, [rows, ibase + j]))
            outv[pl.ds(c * CHUNK + i0, LANES)] = acc

    pltpu.sync_copy(outv, out_hbm.at[pl.ds(wid * BPW, BPW)])


@jax.jit
def _mf(ubig3, ibig3, uoff2, ioff2, utab, itab):
    mesh = plsc.VectorSubcoreMesh(core_axis_name="c", subcore_axis_name="s")
    run = pl.kernel(
        _mf_body,
        out_type=jax.ShapeDtypeStruct((BATCH,), jnp.float32),
        mesh=mesh,
        compiler_params=pltpu.CompilerParams(
            needs_layout_passes=False, use_tc_tiling_on_sc=False),
        scratch_types=[
            pltpu.VMEM((NCHUNKS, CHUNK), jnp.int32),
            pltpu.VMEM((NCHUNKS, CHUNK), jnp.int32),
            pltpu.VMEM((BPW,), jnp.int32),
            pltpu.VMEM((BPW,), jnp.int32),
            pltpu.VMEM((CHUNK, 128), jnp.float32),
            pltpu.VMEM((CHUNK, 128), jnp.float32),
            pltpu.VMEM((BPW,), jnp.float32),
            pltpu.SemaphoreType.DMA,
            pltpu.SemaphoreType.DMA,
        ],
    )
    return run(ubig3, ibig3, uoff2, ioff2, utab, itab)


def kernel(user, item, user_table, item_table):
    user = user.astype(jnp.int32)
    item = item.astype(jnp.int32)
    ubig3 = (user // ROWS_PER_SUPER).reshape(NUM_WORKERS, NCHUNKS, CHUNK)
    ibig3 = (item // ROWS_PER_SUPER).reshape(NUM_WORKERS, NCHUNKS, CHUNK)
    uoff2 = ((user % ROWS_PER_SUPER) * EMB).reshape(NUM_WORKERS, BPW)
    ioff2 = ((item % ROWS_PER_SUPER) * EMB).reshape(NUM_WORKERS, BPW)
    utab = user_table.reshape(-1, 128)
    itab = item_table.reshape(-1, 128)
    return _mf(ubig3, ibig3, uoff2, ioff2, utab, itab)
